# initial kernel scaffold (unmeasured)
import jax
import jax.numpy as jnp
from jax import lax
from jax.experimental import pallas as pl
from jax.experimental.pallas import tpu as pltpu

N_DEV = 16
N_TOK = 2048
D_MODEL = 512
N_EXP = 128
D_FF = 1024
E_LOC = N_EXP // N_DEV
CHUNK = N_TOK // N_DEV


def kernel(x, router_W, route_idx, expert_W):
    def body(x_ref, rw_ref, idx_ref, ew_ref, out_ref,
             acc_ref, rs_buf, rs_send,
             rs_send_sems, rs_recv_sems, ag_send_sems, ag_recv_sems):
        my = lax.axis_index("i")
        left = lax.rem(my - 1 + N_DEV, N_DEV)
        right = lax.rem(my + 1, N_DEV)

        barrier_sem = pltpu.get_barrier_semaphore()
        for nbr in (left, right):
            pl.semaphore_signal(barrier_sem, inc=1, device_id=(nbr,),
                                device_id_type=pl.DeviceIdType.MESH)
        pl.semaphore_wait(barrier_sem, 2)

        xf = x_ref[...]
        scores = jnp.dot(xf, rw_ref[...],
                         preferred_element_type=jnp.float32)
        m = jnp.max(scores, axis=-1, keepdims=True)
        idx0 = idx_ref[:, 0:1]
        idx1 = idx_ref[:, 1:2]
        iota_e = lax.broadcasted_iota(jnp.int32, (N_TOK, N_EXP), 1)
        s0 = jnp.sum(jnp.where(iota_e == idx0, scores, 0.0),
                     axis=-1, keepdims=True)
        s1 = jnp.sum(jnp.where(iota_e == idx1, scores, 0.0),
                     axis=-1, keepdims=True)
        denom = jnp.exp(s0 - m) + jnp.exp(s1 - m)
        gid = my * E_LOC + lax.broadcasted_iota(jnp.int32, (1, E_LOC), 1)
        mask = jnp.logical_or(idx0 == gid, idx1 == gid)
        s_loc = lax.dynamic_slice(scores, (0, my * E_LOC), (N_TOK, E_LOC))
        w = jnp.where(mask, jnp.exp(s_loc - m) / denom, 0.0)

        acc = jnp.zeros((N_TOK, D_FF), jnp.float32)
        for le in range(E_LOC):
            xw = (xf * w[:, le:le + 1]).astype(jnp.bfloat16)
            acc = acc + jnp.dot(xw, ew_ref[le].astype(jnp.bfloat16),
                                preferred_element_type=jnp.float32)
        acc_ref[...] = acc

        for s in range(N_DEV - 1):
            c_send = lax.rem(my - s + N_DEV, N_DEV)
            local_chunk = acc_ref[pl.ds(c_send * CHUNK, CHUNK), :]
            if s == 0:
                rs_send[0] = local_chunk
            else:
                rs_send[0] = rs_buf[s - 1] + local_chunk
            rdma = pltpu.make_async_remote_copy(
                src_ref=rs_send.at[0],
                dst_ref=rs_buf.at[s],
                send_sem=rs_send_sems.at[s],
                recv_sem=rs_recv_sems.at[s],
                device_id=(right,),
                device_id_type=pl.DeviceIdType.MESH,
            )
            rdma.start()
            rdma.wait()

        c_fin = lax.rem(my + 1, N_DEV)
        out_ref[pl.ds(c_fin * CHUNK, CHUNK), :] = (
            rs_buf[N_DEV - 2] + acc_ref[pl.ds(c_fin * CHUNK, CHUNK), :])

        for s in range(N_DEV - 1):
            c = lax.rem(my + 1 - s + N_DEV, N_DEV)
            rdma = pltpu.make_async_remote_copy(
                src_ref=out_ref.at[pl.ds(c * CHUNK, CHUNK), :],
                dst_ref=out_ref.at[pl.ds(c * CHUNK, CHUNK), :],
                send_sem=ag_send_sems.at[s],
                recv_sem=ag_recv_sems.at[s],
                device_id=(right,),
                device_id_type=pl.DeviceIdType.MESH,
            )
            rdma.start()
            rdma.wait()

    return pl.pallas_call(
        body,
        out_shape=jax.ShapeDtypeStruct((N_TOK, D_FF), jnp.float32),
        in_specs=[
            pl.BlockSpec(memory_space=pltpu.VMEM),
            pl.BlockSpec(memory_space=pltpu.VMEM),
            pl.BlockSpec(memory_space=pltpu.VMEM),
            pl.BlockSpec(memory_space=pltpu.VMEM),
        ],
        out_specs=pl.BlockSpec(memory_space=pltpu.VMEM),
        scratch_shapes=[
            pltpu.VMEM((N_TOK, D_FF), jnp.float32),
            pltpu.VMEM((N_DEV - 1, CHUNK, D_FF), jnp.float32),
            pltpu.VMEM((1, CHUNK, D_FF), jnp.float32),
            pltpu.SemaphoreType.DMA((N_DEV - 1,)),
            pltpu.SemaphoreType.DMA((N_DEV - 1,)),
            pltpu.SemaphoreType.DMA((N_DEV - 1,)),
            pltpu.SemaphoreType.DMA((N_DEV - 1,)),
        ],
        compiler_params=pltpu.CompilerParams(collective_id=0),
    )(x, router_W, route_idx, expert_W)


# baseline (device time: 268187 ns/iter reference)
import jax
import jax.numpy as jnp
from jax import lax
from jax.experimental import pallas as pl
from jax.experimental.pallas import tpu as pltpu

N_DEV = 16
N_TOK = 2048
D_MODEL = 512
N_EXP = 128
D_FF = 1024
E_LOC = N_EXP // N_DEV
CHUNK = N_TOK // N_DEV


def kernel(x, router_W, route_idx, expert_W):
    def body(x_ref, rw_ref, idx_ref, ew_ref, out_ref,
             acc_ref, rs_buf, rs_send,
             rs_send_sems, rs_recv_sems, ag_send_sems, ag_recv_sems):
        my = lax.axis_index("i")
        left = lax.rem(my - 1 + N_DEV, N_DEV)
        right = lax.rem(my + 1, N_DEV)

        barrier_sem = pltpu.get_barrier_semaphore()
        for nbr in (left, right):
            pl.semaphore_signal(barrier_sem, inc=1, device_id=(nbr,),
                                device_id_type=pl.DeviceIdType.MESH)
        pl.semaphore_wait(barrier_sem, 2)

        xf = x_ref[...]
        scores = jnp.dot(xf, rw_ref[...],
                         preferred_element_type=jnp.float32)
        m = jnp.max(scores, axis=-1, keepdims=True)
        idx0 = idx_ref[:, 0:1]
        idx1 = idx_ref[:, 1:2]
        iota_e = lax.broadcasted_iota(jnp.int32, (N_TOK, N_EXP), 1)
        s0 = jnp.sum(jnp.where(iota_e == idx0, scores, 0.0),
                     axis=-1, keepdims=True)
        s1 = jnp.sum(jnp.where(iota_e == idx1, scores, 0.0),
                     axis=-1, keepdims=True)
        denom = jnp.exp(s0 - m) + jnp.exp(s1 - m)
        gid = my * E_LOC + lax.broadcasted_iota(jnp.int32, (1, E_LOC), 1)
        w = (jnp.where(idx0 == gid, jnp.exp(s0 - m), 0.0)
             + jnp.where(idx1 == gid, jnp.exp(s1 - m), 0.0)) / denom

        for le in range(E_LOC):
            xw = (xf * w[:, le:le + 1]).astype(jnp.bfloat16)
            contrib = jnp.dot(xw, ew_ref[le].astype(jnp.bfloat16),
                              preferred_element_type=jnp.float32)
            if le == 0:
                acc_ref[...] = contrib
            else:
                acc_ref[...] += contrib

        for s in range(N_DEV - 1):
            c_send = lax.rem(my - s + N_DEV, N_DEV)
            local_chunk = acc_ref[pl.ds(c_send * CHUNK, CHUNK), :]
            if s == 0:
                rs_send[0] = local_chunk
            else:
                rs_send[0] = rs_buf[s - 1] + local_chunk
            rdma = pltpu.make_async_remote_copy(
                src_ref=rs_send.at[0],
                dst_ref=rs_buf.at[s],
                send_sem=rs_send_sems.at[s],
                recv_sem=rs_recv_sems.at[s],
                device_id=(right,),
                device_id_type=pl.DeviceIdType.MESH,
            )
            rdma.start()
            rdma.wait()

        c_fin = lax.rem(my + 1, N_DEV)
        out_ref[pl.ds(c_fin * CHUNK, CHUNK), :] = (
            rs_buf[N_DEV - 2] + acc_ref[pl.ds(c_fin * CHUNK, CHUNK), :])

        for s in range(N_DEV - 1):
            c = lax.rem(my + 1 - s + N_DEV, N_DEV)
            rdma = pltpu.make_async_remote_copy(
                src_ref=out_ref.at[pl.ds(c * CHUNK, CHUNK), :],
                dst_ref=out_ref.at[pl.ds(c * CHUNK, CHUNK), :],
                send_sem=ag_send_sems.at[s],
                recv_sem=ag_recv_sems.at[s],
                device_id=(right,),
                device_id_type=pl.DeviceIdType.MESH,
            )
            rdma.start()
            rdma.wait()

    return pl.pallas_call(
        body,
        out_shape=jax.ShapeDtypeStruct((N_TOK, D_FF), jnp.float32),
        in_specs=[
            pl.BlockSpec(memory_space=pltpu.VMEM),
            pl.BlockSpec(memory_space=pltpu.VMEM),
            pl.BlockSpec(memory_space=pltpu.VMEM),
            pl.BlockSpec(memory_space=pltpu.VMEM),
        ],
        out_specs=pl.BlockSpec(memory_space=pltpu.VMEM),
        scratch_shapes=[
            pltpu.VMEM((N_TOK, D_FF), jnp.float32),
            pltpu.VMEM((N_DEV - 1, CHUNK, D_FF), jnp.float32),
            pltpu.VMEM((1, CHUNK, D_FF), jnp.float32),
            pltpu.SemaphoreType.DMA((N_DEV - 1,)),
            pltpu.SemaphoreType.DMA((N_DEV - 1,)),
            pltpu.SemaphoreType.DMA((N_DEV - 1,)),
            pltpu.SemaphoreType.DMA((N_DEV - 1,)),
        ],
        compiler_params=pltpu.CompilerParams(
            collective_id=0,
            vmem_limit_bytes=100 * 1024 * 1024,
        ),
    )(x, router_W, route_idx, expert_W)


# device time: 184367 ns/iter; 1.4546x vs baseline; 1.4546x over previous
import jax
import jax.numpy as jnp
from jax import lax
from jax.experimental import pallas as pl
from jax.experimental.pallas import tpu as pltpu

N_DEV = 16
N_TOK = 2048
D_MODEL = 512
N_EXP = 128
D_FF = 1024
E_LOC = N_EXP // N_DEV
CHUNK = N_TOK // N_DEV


def kernel(x, router_W, route_idx, expert_W):
    def body(x_ref, rw_ref, idx_ref, ew_ref, out_ref,
             acc_ref, rs_buf, rs_send, ag_buf,
             rs_send_sems, rs_recv_sems, ag_send_sems, ag_recv_sems):
        my = lax.axis_index("i")
        left = lax.rem(my - 1 + N_DEV, N_DEV)
        right = lax.rem(my + 1, N_DEV)

        barrier_sem = pltpu.get_barrier_semaphore()
        for nbr in (left, right):
            pl.semaphore_signal(barrier_sem, inc=1, device_id=(nbr,),
                                device_id_type=pl.DeviceIdType.MESH)
        pl.semaphore_wait(barrier_sem, 2)

        xf = x_ref[...]
        scores = jnp.dot(xf, rw_ref[...],
                         preferred_element_type=jnp.float32)
        m = jnp.max(scores, axis=-1, keepdims=True)
        idx0 = idx_ref[:, 0:1]
        idx1 = idx_ref[:, 1:2]
        iota_e = lax.broadcasted_iota(jnp.int32, (N_TOK, N_EXP), 1)
        s0 = jnp.sum(jnp.where(iota_e == idx0, scores, 0.0),
                     axis=-1, keepdims=True)
        s1 = jnp.sum(jnp.where(iota_e == idx1, scores, 0.0),
                     axis=-1, keepdims=True)
        denom = jnp.exp(s0 - m) + jnp.exp(s1 - m)
        gid = my * E_LOC + lax.broadcasted_iota(jnp.int32, (1, E_LOC), 1)
        w = (jnp.where(idx0 == gid, jnp.exp(s0 - m), 0.0)
             + jnp.where(idx1 == gid, jnp.exp(s1 - m), 0.0)) / denom

        for le in range(E_LOC):
            xw = (xf * w[:, le:le + 1]).astype(jnp.bfloat16)
            contrib = jnp.dot(xw, ew_ref[le].astype(jnp.bfloat16),
                              preferred_element_type=jnp.float32)
            if le == 0:
                acc_ref[...] = contrib
            else:
                acc_ref[...] += contrib

        for s in range(N_DEV - 1):
            c_send = lax.rem(my - s + N_DEV, N_DEV)
            local_chunk = acc_ref[pl.ds(c_send * CHUNK, CHUNK), :]
            if s == 0:
                rs_send[0] = local_chunk.astype(jnp.bfloat16)
            else:
                rs_send[0] = (rs_buf[s - 1].astype(jnp.float32)
                              + local_chunk).astype(jnp.bfloat16)
            rdma = pltpu.make_async_remote_copy(
                src_ref=rs_send.at[0],
                dst_ref=rs_buf.at[s],
                send_sem=rs_send_sems.at[s],
                recv_sem=rs_recv_sems.at[s],
                device_id=(right,),
                device_id_type=pl.DeviceIdType.MESH,
            )
            rdma.start()
            rdma.wait()

        c_fin = lax.rem(my + 1, N_DEV)
        ag_buf[pl.ds(c_fin * CHUNK, CHUNK), :] = (
            rs_buf[N_DEV - 2].astype(jnp.float32)
            + acc_ref[pl.ds(c_fin * CHUNK, CHUNK), :]).astype(jnp.bfloat16)

        for s in range(N_DEV - 1):
            c = lax.rem(my + 1 - s + N_DEV, N_DEV)
            rdma = pltpu.make_async_remote_copy(
                src_ref=ag_buf.at[pl.ds(c * CHUNK, CHUNK), :],
                dst_ref=ag_buf.at[pl.ds(c * CHUNK, CHUNK), :],
                send_sem=ag_send_sems.at[s],
                recv_sem=ag_recv_sems.at[s],
                device_id=(right,),
                device_id_type=pl.DeviceIdType.MESH,
            )
            rdma.start()
            rdma.wait()

        out_ref[...] = ag_buf[...].astype(jnp.float32)

    return pl.pallas_call(
        body,
        out_shape=jax.ShapeDtypeStruct((N_TOK, D_FF), jnp.float32),
        in_specs=[
            pl.BlockSpec(memory_space=pltpu.VMEM),
            pl.BlockSpec(memory_space=pltpu.VMEM),
            pl.BlockSpec(memory_space=pltpu.VMEM),
            pl.BlockSpec(memory_space=pltpu.VMEM),
        ],
        out_specs=pl.BlockSpec(memory_space=pltpu.VMEM),
        scratch_shapes=[
            pltpu.VMEM((N_TOK, D_FF), jnp.float32),
            pltpu.VMEM((N_DEV - 1, CHUNK, D_FF), jnp.bfloat16),
            pltpu.VMEM((1, CHUNK, D_FF), jnp.bfloat16),
            pltpu.VMEM((N_TOK, D_FF), jnp.bfloat16),
            pltpu.SemaphoreType.DMA((N_DEV - 1,)),
            pltpu.SemaphoreType.DMA((N_DEV - 1,)),
            pltpu.SemaphoreType.DMA((N_DEV - 1,)),
            pltpu.SemaphoreType.DMA((N_DEV - 1,)),
        ],
        compiler_params=pltpu.CompilerParams(
            collective_id=0,
            vmem_limit_bytes=100 * 1024 * 1024,
        ),
    )(x, router_W, route_idx, expert_W)
